# Initial kernel scaffold; baseline (speedup 1.0000x reference)
import jax
import jax.numpy as jnp
from jax.experimental import pallas as pl


def kernel(x, edge_index, W1, b1, W2, b2):
    raise NotImplementedError("write your pallas kernel here")


# speedup vs baseline: 191.0821x; 191.0821x over previous
import functools

import jax
import jax.numpy as jnp
from jax import lax
from jax.experimental import pallas as pl
from jax.experimental.pallas import tpu as pltpu
from jax.experimental.pallas import tpu_sc as plsc

N = 100000
E = 6400000
NPAD = 100352
SL = NPAD // 16
ROWS = E // 128
ROWS_PER_W = 1568
LAST_ROWS = ROWS - 31 * ROWS_PER_W
CH = 16
NCH_FULL = ROWS_PER_W // CH
NCH_LAST = LAST_ROWS // CH

_mesh = plsc.VectorSubcoreMesh(core_axis_name="c", subcore_axis_name="s")
_sc_params = pltpu.CompilerParams(needs_layout_passes=False)


def _zero_acc_slice(slice_v, acc_sh, s):
    def zbody(i, _):
        slice_v[pl.ds(i * 16, 16)] = jnp.zeros((16,), jnp.float32)
        return 0
    lax.fori_loop(0, SL // 16, zbody, 0)
    pltpu.sync_copy(slice_v, acc_sh.at[pl.ds(s * SL, SL)])


def _writeback(slice_v, acc_sh, out_hbm, c, s):
    pltpu.sync_copy(acc_sh.at[pl.ds(s * SL, SL)], slice_v)
    pltpu.sync_copy(slice_v, out_hbm.at[c, pl.ds(s * SL, SL)])


@functools.partial(
    pl.kernel,
    out_type=jax.ShapeDtypeStruct((2, NPAD), jnp.float32),
    mesh=_mesh,
    scratch_types=[
        pltpu.VMEM((CH, 128), jnp.int32),
        pltpu.VMEM((128,), jnp.float32),
        pltpu.VMEM((SL,), jnp.float32),
        pltpu.VMEM_SHARED((NPAD,), jnp.float32),
    ],
    compiler_params=_sc_params,
)
def _deg_kernel(edges_hbm, out_hbm, dst_v, ones_v, slice_v, acc_sh):
    c = lax.axis_index("c")
    s = lax.axis_index("s")
    wid = c * 16 + s
    for l in range(8):
        ones_v[pl.ds(l * 16, 16)] = jnp.ones((16,), jnp.float32)
    _zero_acc_slice(slice_v, acc_sh, s)
    plsc.subcore_barrier()

    base_row = wid * ROWS_PER_W
    n_chunks = jnp.where(wid == 31, NCH_LAST, NCH_FULL)

    def chunk(ci, _):
        r0 = base_row + ci * CH
        pltpu.sync_copy(edges_hbm.at[1, pl.ds(r0, CH), :], dst_v)
        for j in range(CH):
            pltpu.sync_copy(ones_v, acc_sh.at[dst_v.at[j]], add=True)
        return 0

    lax.fori_loop(0, n_chunks, chunk, 0)
    plsc.subcore_barrier()
    _writeback(slice_v, acc_sh, out_hbm, c, s)


@functools.partial(
    pl.kernel,
    out_type=jax.ShapeDtypeStruct((2, NPAD), jnp.float32),
    mesh=_mesh,
    scratch_types=[
        pltpu.VMEM((CH, 128), jnp.int32),
        pltpu.VMEM((CH, 128), jnp.int32),
        pltpu.VMEM((CH, 128), jnp.float32),
        pltpu.VMEM((NPAD,), jnp.float32),
        pltpu.VMEM((SL,), jnp.float32),
        pltpu.VMEM_SHARED((NPAD,), jnp.float32),
    ],
    compiler_params=_sc_params,
)
def _agg_kernel(edges_hbm, u_hbm, out_hbm, src_v, dst_v, vals_v, u_v,
                slice_v, acc_sh):
    c = lax.axis_index("c")
    s = lax.axis_index("s")
    wid = c * 16 + s
    pltpu.sync_copy(u_hbm, u_v)
    _zero_acc_slice(slice_v, acc_sh, s)
    plsc.subcore_barrier()

    base_row = wid * ROWS_PER_W
    n_chunks = jnp.where(wid == 31, NCH_LAST, NCH_FULL)

    def chunk(ci, _):
        r0 = base_row + ci * CH
        pltpu.sync_copy(edges_hbm.at[0, pl.ds(r0, CH), :], src_v)
        pltpu.sync_copy(edges_hbm.at[1, pl.ds(r0, CH), :], dst_v)

        def grow(j, _):
            for l in range(8):
                idx = src_v[j, pl.ds(l * 16, 16)]
                vals_v[j, pl.ds(l * 16, 16)] = plsc.load_gather(u_v, [idx])
            return 0

        lax.fori_loop(0, CH, grow, 0)
        for j in range(CH):
            pltpu.sync_copy(vals_v.at[j], acc_sh.at[dst_v.at[j]], add=True)
        return 0

    lax.fori_loop(0, n_chunks, chunk, 0)
    plsc.subcore_barrier()
    _writeback(slice_v, acc_sh, out_hbm, c, s)


def _dinv_u1_body(degp_ref, x_ref, dinv_ref, u1_ref):
    deg = degp_ref[0] + degp_ref[1] + 1.0
    dinv = lax.rsqrt(deg)
    dinv_ref[...] = dinv
    u1_ref[...] = x_ref[...] * dinv


def _mid_body(t1p_ref, dinv_ref, x_ref, w1_ref, b1_ref, w2_ref,
              h2_ref, u2_ref):
    dinv = dinv_ref[...]
    a = dinv * (t1p_ref[0] + t1p_ref[1]) + x_ref[...] * dinv * dinv
    h2 = jnp.zeros_like(a)
    for j in range(3):
        h2 = h2 + w2_ref[0, j] * jnp.maximum(w1_ref[j, 0] * a + b1_ref[j], 0.0)
    h2_ref[...] = h2
    u2_ref[...] = h2 * dinv


def _final_body(t2p_ref, dinv_ref, h2_ref, b2_ref, out_ref):
    dinv = dinv_ref[...]
    o = dinv * (t2p_ref[0] + t2p_ref[1]) + h2_ref[...] * dinv * dinv + b2_ref[0]
    idx = (lax.broadcasted_iota(jnp.int32, o.shape, 0) * 128
           + lax.broadcasted_iota(jnp.int32, o.shape, 1))
    o = jnp.where(idx < N, o, -jnp.inf)
    m = jnp.max(o)
    e = jnp.exp(o - m)
    out_ref[...] = e / jnp.sum(e)


def _smem_spec():
    return pl.BlockSpec(memory_space=pltpu.SMEM)


def kernel(x, edge_index, W1, b1, W2, b2):
    edges3d = edge_index.reshape(2, ROWS, 128)
    x_pad = jnp.pad(x[:, 0], (0, NPAD - N)).reshape(784, 128)

    deg_p = _deg_kernel(edges3d)

    dinv, u1 = pl.pallas_call(
        _dinv_u1_body,
        out_shape=(jax.ShapeDtypeStruct((784, 128), jnp.float32),
                   jax.ShapeDtypeStruct((784, 128), jnp.float32)),
    )(deg_p.reshape(2, 784, 128), x_pad)

    t1_p = _agg_kernel(edges3d, u1.reshape(NPAD))

    h2, u2 = pl.pallas_call(
        _mid_body,
        in_specs=[pl.BlockSpec(), pl.BlockSpec(), pl.BlockSpec(),
                  _smem_spec(), _smem_spec(), _smem_spec()],
        out_shape=(jax.ShapeDtypeStruct((784, 128), jnp.float32),
                   jax.ShapeDtypeStruct((784, 128), jnp.float32)),
    )(t1_p.reshape(2, 784, 128), dinv, x_pad, W1, b1, W2)

    t2_p = _agg_kernel(edges3d, u2.reshape(NPAD))

    out = pl.pallas_call(
        _final_body,
        in_specs=[pl.BlockSpec(), pl.BlockSpec(), pl.BlockSpec(),
                  _smem_spec()],
        out_shape=jax.ShapeDtypeStruct((784, 128), jnp.float32),
    )(t2_p.reshape(2, 784, 128), dinv, h2, b2)

    return out.reshape(NPAD)[:N].reshape(N, 1)


# speedup vs baseline: 261.6609x; 1.3694x over previous
import functools

import jax
import jax.numpy as jnp
from jax import lax
from jax.experimental import pallas as pl
from jax.experimental.pallas import tpu as pltpu
from jax.experimental.pallas import tpu_sc as plsc

N = 100000
E = 6400000
NPAD = 100352
SL = NPAD // 16
CHE = 2048
EPW = 200704
NCH_FULL = EPW // CHE
NCH_LAST = (E - 31 * EPW) // CHE

_mesh = plsc.VectorSubcoreMesh(core_axis_name="c", subcore_axis_name="s")
_sc_params = pltpu.CompilerParams(needs_layout_passes=False)


def _zero_acc_slice(slice_v, acc_sh, s):
    def zbody(i, _):
        slice_v[pl.ds(i * 16, 16)] = jnp.zeros((16,), jnp.float32)
        return 0
    lax.fori_loop(0, SL // 16, zbody, 0)
    pltpu.sync_copy(slice_v, acc_sh.at[pl.ds(s * SL, SL)])


def _writeback(slice_v, acc_sh, out_hbm, c, s):
    pltpu.sync_copy(acc_sh.at[pl.ds(s * SL, SL)], slice_v)
    pltpu.sync_copy(slice_v, out_hbm.at[c, pl.ds(s * SL, SL)])


@functools.partial(
    pl.kernel,
    out_type=jax.ShapeDtypeStruct((2, NPAD), jnp.float32),
    mesh=_mesh,
    scratch_types=[
        pltpu.VMEM((CHE,), jnp.int32),
        pltpu.VMEM((CHE,), jnp.float32),
        pltpu.VMEM((SL,), jnp.float32),
        pltpu.VMEM_SHARED((NPAD,), jnp.float32),
    ],
    compiler_params=_sc_params,
)
def _deg_kernel(edges_hbm, out_hbm, dst_v, ones_v, slice_v, acc_sh):
    c = lax.axis_index("c")
    s = lax.axis_index("s")
    wid = c * 16 + s

    def obody(i, _):
        ones_v[pl.ds(i * 16, 16)] = jnp.ones((16,), jnp.float32)
        return 0
    lax.fori_loop(0, CHE // 16, obody, 0)
    _zero_acc_slice(slice_v, acc_sh, s)
    plsc.subcore_barrier()

    base_e = wid * EPW
    n_chunks = jnp.where(wid == 31, NCH_LAST, NCH_FULL)

    def chunk(ci, _):
        e0 = base_e + ci * CHE
        pltpu.sync_copy(edges_hbm.at[1, pl.ds(e0, CHE)], dst_v)
        pltpu.sync_copy(ones_v, acc_sh.at[dst_v], add=True)
        return 0

    lax.fori_loop(0, n_chunks, chunk, 0)
    plsc.subcore_barrier()
    _writeback(slice_v, acc_sh, out_hbm, c, s)


@functools.partial(
    pl.kernel,
    out_type=jax.ShapeDtypeStruct((2, NPAD), jnp.float32),
    mesh=_mesh,
    scratch_types=[
        pltpu.VMEM((CHE,), jnp.int32),
        pltpu.VMEM((CHE,), jnp.int32),
        pltpu.VMEM((CHE,), jnp.float32),
        pltpu.VMEM((NPAD,), jnp.float32),
        pltpu.VMEM((SL,), jnp.float32),
        pltpu.VMEM_SHARED((NPAD,), jnp.float32),
    ],
    compiler_params=_sc_params,
)
def _agg_kernel(edges_hbm, u_hbm, out_hbm, src_v, dst_v, vals_v, u_v,
                slice_v, acc_sh):
    c = lax.axis_index("c")
    s = lax.axis_index("s")
    wid = c * 16 + s
    pltpu.sync_copy(u_hbm, u_v)
    _zero_acc_slice(slice_v, acc_sh, s)
    plsc.subcore_barrier()

    base_e = wid * EPW
    n_chunks = jnp.where(wid == 31, NCH_LAST, NCH_FULL)

    def chunk(ci, _):
        e0 = base_e + ci * CHE
        pltpu.sync_copy(edges_hbm.at[0, pl.ds(e0, CHE)], src_v)
        pltpu.sync_copy(edges_hbm.at[1, pl.ds(e0, CHE)], dst_v)

        def grow(i, _):
            idx = src_v[pl.ds(i * 16, 16)]
            vals_v[pl.ds(i * 16, 16)] = plsc.load_gather(u_v, [idx])
            return 0

        lax.fori_loop(0, CHE // 16, grow, 0)
        pltpu.sync_copy(vals_v, acc_sh.at[dst_v], add=True)
        return 0

    lax.fori_loop(0, n_chunks, chunk, 0)
    plsc.subcore_barrier()
    _writeback(slice_v, acc_sh, out_hbm, c, s)


def _dinv_u1_body(degp_ref, x_ref, dinv_ref, u1_ref):
    deg = degp_ref[0] + degp_ref[1] + 1.0
    dinv = lax.rsqrt(deg)
    dinv_ref[...] = dinv
    u1_ref[...] = x_ref[...] * dinv


def _mid_body(t1p_ref, dinv_ref, x_ref, w1_ref, b1_ref, w2_ref,
              h2_ref, u2_ref):
    dinv = dinv_ref[...]
    a = dinv * (t1p_ref[0] + t1p_ref[1]) + x_ref[...] * dinv * dinv
    h2 = jnp.zeros_like(a)
    for j in range(3):
        h2 = h2 + w2_ref[0, j] * jnp.maximum(w1_ref[j, 0] * a + b1_ref[j], 0.0)
    h2_ref[...] = h2
    u2_ref[...] = h2 * dinv


def _final_body(t2p_ref, dinv_ref, h2_ref, b2_ref, out_ref):
    dinv = dinv_ref[...]
    o = dinv * (t2p_ref[0] + t2p_ref[1]) + h2_ref[...] * dinv * dinv + b2_ref[0]
    idx = (lax.broadcasted_iota(jnp.int32, o.shape, 0) * 128
           + lax.broadcasted_iota(jnp.int32, o.shape, 1))
    o = jnp.where(idx < N, o, -jnp.inf)
    m = jnp.max(o)
    e = jnp.exp(o - m)
    out_ref[...] = e / jnp.sum(e)


def _smem_spec():
    return pl.BlockSpec(memory_space=pltpu.SMEM)


def kernel(x, edge_index, W1, b1, W2, b2):
    x_pad = jnp.pad(x[:, 0], (0, NPAD - N)).reshape(784, 128)

    deg_p = _deg_kernel(edge_index)

    dinv, u1 = pl.pallas_call(
        _dinv_u1_body,
        out_shape=(jax.ShapeDtypeStruct((784, 128), jnp.float32),
                   jax.ShapeDtypeStruct((784, 128), jnp.float32)),
    )(deg_p.reshape(2, 784, 128), x_pad)

    t1_p = _agg_kernel(edge_index, u1.reshape(NPAD))

    h2, u2 = pl.pallas_call(
        _mid_body,
        in_specs=[pl.BlockSpec(), pl.BlockSpec(), pl.BlockSpec(),
                  _smem_spec(), _smem_spec(), _smem_spec()],
        out_shape=(jax.ShapeDtypeStruct((784, 128), jnp.float32),
                   jax.ShapeDtypeStruct((784, 128), jnp.float32)),
    )(t1_p.reshape(2, 784, 128), dinv, x_pad, W1, b1, W2)

    t2_p = _agg_kernel(edge_index, u2.reshape(NPAD))

    out = pl.pallas_call(
        _final_body,
        in_specs=[pl.BlockSpec(), pl.BlockSpec(), pl.BlockSpec(),
                  _smem_spec()],
        out_shape=jax.ShapeDtypeStruct((784, 128), jnp.float32),
    )(t2_p.reshape(2, 784, 128), dinv, h2, b2)

    return out.reshape(NPAD)[:N].reshape(N, 1)


# speedup vs baseline: 720.7725x; 2.7546x over previous
import functools

import jax
import jax.numpy as jnp
from jax import lax
from jax.experimental import pallas as pl
from jax.experimental.pallas import tpu as pltpu
from jax.experimental.pallas import tpu_sc as plsc

N = 100000
E = 6400000
NPAD = 100352
SL = NPAD // 16
CHE = 2048
EPW = 200704
NCH_FULL = EPW // CHE
NCH_LAST = (E - 31 * EPW) // CHE

_mesh = plsc.VectorSubcoreMesh(core_axis_name="c", subcore_axis_name="s")
_sc_params = pltpu.CompilerParams(needs_layout_passes=False)


def _zero_acc_slice(slice_v, acc_sh, s):
    def zbody(i, _):
        slice_v[pl.ds(i * 16, 16)] = jnp.zeros((16,), jnp.float32)
        return 0
    lax.fori_loop(0, SL // 16, zbody, 0)
    pltpu.sync_copy(slice_v, acc_sh.at[pl.ds(s * SL, SL)])


def _writeback(slice_v, acc_sh, out_hbm, c, s):
    pltpu.sync_copy(acc_sh.at[pl.ds(s * SL, SL)], slice_v)
    pltpu.sync_copy(slice_v, out_hbm.at[c, pl.ds(s * SL, SL)])


@functools.partial(
    pl.kernel,
    out_type=jax.ShapeDtypeStruct((2, NPAD), jnp.float32),
    mesh=_mesh,
    scratch_types=[
        pltpu.VMEM((CHE,), jnp.int32),
        pltpu.VMEM((CHE,), jnp.int32),
        pltpu.VMEM((CHE,), jnp.int32),
        pltpu.VMEM((CHE,), jnp.int32),
        pltpu.VMEM((CHE,), jnp.float32),
        pltpu.VMEM((SL,), jnp.float32),
        pltpu.VMEM_SHARED((NPAD,), jnp.float32),
        pltpu.SemaphoreType.DMA,
        pltpu.SemaphoreType.DMA,
        pltpu.SemaphoreType.DMA,
        pltpu.SemaphoreType.DMA,
    ],
    compiler_params=_sc_params,
)
def _deg_kernel(edges_hbm, out_hbm, dstS0, dstS1, dstX0, dstX1, ones_v,
                slice_v, acc_sh, dma0, dma1, sc0, sc1):
    c = lax.axis_index("c")
    s = lax.axis_index("s")
    wid = c * 16 + s

    def obody(i, _):
        ones_v[pl.ds(i * 16, 16)] = jnp.ones((16,), jnp.float32)
        return 0
    lax.fori_loop(0, CHE // 16, obody, 0)
    _zero_acc_slice(slice_v, acc_sh, s)
    plsc.subcore_barrier()

    base_e = wid * EPW
    nc = jnp.where(wid == 31, NCH_LAST, NCH_FULL)
    bufs = ((dstS0, dstX0, dma0, sc0), (dstS1, dstX1, dma1, sc1))

    def fire_dma(ci, P):
        dstS, _, dma, _ = bufs[P]
        e0 = base_e + ci * CHE
        pltpu.async_copy(edges_hbm.at[1, pl.ds(e0, CHE)], dstS, dma)

    def process(ci, P):
        dstS, dstX, dma, sc = bufs[P]
        pltpu.make_async_copy(edges_hbm.at[1, pl.ds(0, CHE)], dstS, dma).wait()

        @pl.when(ci >= 2)
        def _():
            pltpu.make_async_copy(ones_v, acc_sh.at[dstX], sc).wait()

        def cbody(i, _):
            for l in range(4):
                o = i * 64 + l * 16
                dstX[pl.ds(o, 16)] = dstS[pl.ds(o, 16)]
            return 0

        lax.fori_loop(0, CHE // 64, cbody, 0)

        @pl.when(ci + 2 < nc)
        def _():
            fire_dma(ci + 2, P)

        pltpu.async_copy(ones_v, acc_sh.at[dstX], sc, add=True)

    fire_dma(0, 0)
    fire_dma(1, 1)

    def pair(k, carry):
        process(k * 2, 0)
        process(k * 2 + 1, 1)
        return 0

    lax.fori_loop(0, nc // 2, pair, 0)

    @pl.when(nc % 2 == 1)
    def _():
        process(nc - 1, 0)

    pltpu.make_async_copy(ones_v, acc_sh.at[dstX0], sc0).wait()
    pltpu.make_async_copy(ones_v, acc_sh.at[dstX1], sc1).wait()
    plsc.subcore_barrier()
    _writeback(slice_v, acc_sh, out_hbm, c, s)


@functools.partial(
    pl.kernel,
    out_type=jax.ShapeDtypeStruct((2, NPAD), jnp.float32),
    mesh=_mesh,
    scratch_types=[
        pltpu.VMEM((CHE,), jnp.int32),
        pltpu.VMEM((CHE,), jnp.int32),
        pltpu.VMEM((CHE,), jnp.int32),
        pltpu.VMEM((CHE,), jnp.int32),
        pltpu.VMEM((CHE,), jnp.int32),
        pltpu.VMEM((CHE,), jnp.int32),
        pltpu.VMEM((CHE,), jnp.float32),
        pltpu.VMEM((CHE,), jnp.float32),
        pltpu.VMEM((NPAD,), jnp.float32),
        pltpu.VMEM((SL,), jnp.float32),
        pltpu.VMEM_SHARED((NPAD,), jnp.float32),
        pltpu.SemaphoreType.DMA,
        pltpu.SemaphoreType.DMA,
        pltpu.SemaphoreType.DMA,
        pltpu.SemaphoreType.DMA,
    ],
    compiler_params=_sc_params,
)
def _agg_kernel(edges_hbm, u_hbm, out_hbm, src0, src1, dstS0, dstS1,
                dstX0, dstX1, vals0, vals1, u_v, slice_v, acc_sh,
                dma0, dma1, sc0, sc1):
    c = lax.axis_index("c")
    s = lax.axis_index("s")
    wid = c * 16 + s
    pltpu.sync_copy(u_hbm, u_v)
    _zero_acc_slice(slice_v, acc_sh, s)
    plsc.subcore_barrier()

    base_e = wid * EPW
    nc = jnp.where(wid == 31, NCH_LAST, NCH_FULL)
    bufs = ((src0, dstS0, dstX0, vals0, dma0, sc0),
            (src1, dstS1, dstX1, vals1, dma1, sc1))

    def fire_dma(ci, P):
        src_v, dstS, _, _, dma, _ = bufs[P]
        e0 = base_e + ci * CHE
        pltpu.async_copy(edges_hbm.at[0, pl.ds(e0, CHE)], src_v, dma)
        pltpu.async_copy(edges_hbm.at[1, pl.ds(e0, CHE)], dstS, dma)

    def process(ci, P):
        src_v, dstS, dstX, vals, dma, sc = bufs[P]
        pltpu.make_async_copy(edges_hbm.at[0, pl.ds(0, CHE)], src_v, dma).wait()
        pltpu.make_async_copy(edges_hbm.at[1, pl.ds(0, CHE)], dstS, dma).wait()

        @pl.when(ci >= 2)
        def _():
            pltpu.make_async_copy(vals, acc_sh.at[dstX], sc).wait()

        def gbody(i, _):
            for l in range(4):
                o = i * 64 + l * 16
                idx = src_v[pl.ds(o, 16)]
                vals[pl.ds(o, 16)] = plsc.load_gather(u_v, [idx])
                dstX[pl.ds(o, 16)] = dstS[pl.ds(o, 16)]
            return 0

        lax.fori_loop(0, CHE // 64, gbody, 0)

        @pl.when(ci + 2 < nc)
        def _():
            fire_dma(ci + 2, P)

        pltpu.async_copy(vals, acc_sh.at[dstX], sc, add=True)

    fire_dma(0, 0)
    fire_dma(1, 1)

    def pair(k, carry):
        process(k * 2, 0)
        process(k * 2 + 1, 1)
        return 0

    lax.fori_loop(0, nc // 2, pair, 0)

    @pl.when(nc % 2 == 1)
    def _():
        process(nc - 1, 0)

    pltpu.make_async_copy(vals0, acc_sh.at[dstX0], sc0).wait()
    pltpu.make_async_copy(vals1, acc_sh.at[dstX1], sc1).wait()
    plsc.subcore_barrier()
    _writeback(slice_v, acc_sh, out_hbm, c, s)


def _dinv_u1_body(degp_ref, x_ref, dinv_ref, u1_ref):
    deg = degp_ref[0] + degp_ref[1] + 1.0
    dinv = lax.rsqrt(deg)
    dinv_ref[...] = dinv
    u1_ref[...] = x_ref[...] * dinv


def _mid_body(t1p_ref, dinv_ref, x_ref, w1_ref, b1_ref, w2_ref,
              h2_ref, u2_ref):
    dinv = dinv_ref[...]
    a = dinv * (t1p_ref[0] + t1p_ref[1]) + x_ref[...] * dinv * dinv
    h2 = jnp.zeros_like(a)
    for j in range(3):
        h2 = h2 + w2_ref[0, j] * jnp.maximum(w1_ref[j, 0] * a + b1_ref[j], 0.0)
    h2_ref[...] = h2
    u2_ref[...] = h2 * dinv


def _final_body(t2p_ref, dinv_ref, h2_ref, b2_ref, out_ref):
    dinv = dinv_ref[...]
    o = dinv * (t2p_ref[0] + t2p_ref[1]) + h2_ref[...] * dinv * dinv + b2_ref[0]
    idx = (lax.broadcasted_iota(jnp.int32, o.shape, 0) * 128
           + lax.broadcasted_iota(jnp.int32, o.shape, 1))
    o = jnp.where(idx < N, o, -jnp.inf)
    m = jnp.max(o)
    e = jnp.exp(o - m)
    out_ref[...] = e / jnp.sum(e)


def _smem_spec():
    return pl.BlockSpec(memory_space=pltpu.SMEM)


def kernel(x, edge_index, W1, b1, W2, b2):
    x_pad = jnp.pad(x[:, 0], (0, NPAD - N)).reshape(784, 128)

    deg_p = _deg_kernel(edge_index)

    dinv, u1 = pl.pallas_call(
        _dinv_u1_body,
        out_shape=(jax.ShapeDtypeStruct((784, 128), jnp.float32),
                   jax.ShapeDtypeStruct((784, 128), jnp.float32)),
    )(deg_p.reshape(2, 784, 128), x_pad)

    t1_p = _agg_kernel(edge_index, u1.reshape(NPAD))

    h2, u2 = pl.pallas_call(
        _mid_body,
        in_specs=[pl.BlockSpec(), pl.BlockSpec(), pl.BlockSpec(),
                  _smem_spec(), _smem_spec(), _smem_spec()],
        out_shape=(jax.ShapeDtypeStruct((784, 128), jnp.float32),
                   jax.ShapeDtypeStruct((784, 128), jnp.float32)),
    )(t1_p.reshape(2, 784, 128), dinv, x_pad, W1, b1, W2)

    t2_p = _agg_kernel(edge_index, u2.reshape(NPAD))

    out = pl.pallas_call(
        _final_body,
        in_specs=[pl.BlockSpec(), pl.BlockSpec(), pl.BlockSpec(),
                  _smem_spec()],
        out_shape=jax.ShapeDtypeStruct((784, 128), jnp.float32),
    )(t2_p.reshape(2, 784, 128), dinv, h2, b2)

    return out.reshape(NPAD)[:N].reshape(N, 1)


# speedup vs baseline: 763.4340x; 1.0592x over previous
import functools

import jax
import jax.numpy as jnp
from jax import lax
from jax.experimental import pallas as pl
from jax.experimental.pallas import tpu as pltpu
from jax.experimental.pallas import tpu_sc as plsc

N = 100000
E = 6400000
NPAD = 100352
SL = NPAD // 16
CHE = 2048
EPW = 200704
NCH_FULL = EPW // CHE
NCH_LAST = (E - 31 * EPW) // CHE

_mesh = plsc.VectorSubcoreMesh(core_axis_name="c", subcore_axis_name="s")
_sc_params = pltpu.CompilerParams(needs_layout_passes=False)


def _fill(buf, n, value):
    def body(i, _):
        buf[pl.ds(i * 16, 16)] = jnp.full((16,), value, jnp.float32)
        return 0
    lax.fori_loop(0, n // 16, body, 0)


def _zero_acc_slice(zbuf, acc_sh, s):
    base = s * SL
    for i in range(3):
        pltpu.sync_copy(zbuf, acc_sh.at[pl.ds(base + i * CHE, CHE)])
    pltpu.sync_copy(zbuf.at[pl.ds(0, 128)], acc_sh.at[pl.ds(base + 3 * CHE, 128)])


def _writeback(tbuf, acc_sh, out_hbm, c, s):
    base = s * SL
    for i in range(3):
        pltpu.sync_copy(acc_sh.at[pl.ds(base + i * CHE, CHE)], tbuf)
        pltpu.sync_copy(tbuf, out_hbm.at[c, pl.ds(base + i * CHE, CHE)])
    pltpu.sync_copy(acc_sh.at[pl.ds(base + 3 * CHE, 128)], tbuf.at[pl.ds(0, 128)])
    pltpu.sync_copy(tbuf.at[pl.ds(0, 128)], out_hbm.at[c, pl.ds(base + 3 * CHE, 128)])


@functools.partial(
    pl.kernel,
    out_type=jax.ShapeDtypeStruct((2, NPAD), jnp.float32),
    mesh=_mesh,
    scratch_types=[
        pltpu.VMEM((CHE,), jnp.int32),
        pltpu.VMEM((CHE,), jnp.int32),
        pltpu.VMEM((CHE,), jnp.int32),
        pltpu.VMEM((CHE,), jnp.int32),
        pltpu.VMEM((CHE,), jnp.float32),
        pltpu.VMEM((CHE,), jnp.float32),
        pltpu.VMEM_SHARED((NPAD,), jnp.float32),
        pltpu.SemaphoreType.DMA,
        pltpu.SemaphoreType.DMA,
        pltpu.SemaphoreType.DMA,
        pltpu.SemaphoreType.DMA,
        pltpu.SemaphoreType.DMA,
        pltpu.SemaphoreType.DMA,
        pltpu.SemaphoreType.DMA,
        pltpu.SemaphoreType.DMA,
    ],
    compiler_params=_sc_params,
)
def _deg_kernel(edges_hbm, out_hbm, dst0, dst1, dst2, dst3, ones_v, zbuf,
                acc_sh, dma0, dma1, dma2, dma3, sc0, sc1, sc2, sc3):
    c = lax.axis_index("c")
    s = lax.axis_index("s")
    wid = c * 16 + s
    _fill(ones_v, CHE, 1.0)
    _fill(zbuf, CHE, 0.0)
    _zero_acc_slice(zbuf, acc_sh, s)
    plsc.subcore_barrier()

    base_e = wid * EPW
    nc = jnp.where(wid == 31, NCH_LAST, NCH_FULL)
    dsts = (dst0, dst1, dst2, dst3)
    dmas = (dma0, dma1, dma2, dma3)
    scs = (sc0, sc1, sc2, sc3)

    def fire_dma(ci, P):
        e0 = base_e + ci * CHE
        pltpu.async_copy(edges_hbm.at[1, pl.ds(e0, CHE)], dsts[P], dmas[P])

    def drain_sc(P):
        pltpu.make_async_copy(ones_v, acc_sh.at[dsts[P]], scs[P]).wait()

    def process(ci, P):
        Q = (P + 2) % 4
        pltpu.make_async_copy(edges_hbm.at[1, pl.ds(0, CHE)], dsts[P],
                              dmas[P]).wait()

        @pl.when(ci >= 2)
        def _():
            drain_sc(Q)

        @pl.when(ci + 2 < nc)
        def _():
            fire_dma(ci + 2, Q)

        pltpu.async_copy(ones_v, acc_sh.at[dsts[P]], scs[P], add=True)

    fire_dma(0, 0)
    fire_dma(1, 1)

    def quad(k, carry):
        for t in range(4):
            process(k * 4 + t, t)
        return 0

    lax.fori_loop(0, nc // 4, quad, 0)
    for t in range(3):
        @pl.when(t < nc % 4)
        def _(t=t):
            process((nc // 4) * 4 + t, t)

    @pl.when(wid != 31)
    def _():
        drain_sc(0)
        drain_sc(1)

    @pl.when(wid == 31)
    def _():
        drain_sc(1)
        drain_sc(2)

    plsc.subcore_barrier()
    _writeback(zbuf, acc_sh, out_hbm, c, s)


@functools.partial(
    pl.kernel,
    out_type=jax.ShapeDtypeStruct((2, NPAD), jnp.float32),
    mesh=_mesh,
    scratch_types=[
        pltpu.VMEM((CHE,), jnp.int32),
        pltpu.VMEM((CHE,), jnp.int32),
        pltpu.VMEM((CHE,), jnp.int32),
        pltpu.VMEM((CHE,), jnp.int32),
        pltpu.VMEM((CHE,), jnp.int32),
        pltpu.VMEM((CHE,), jnp.int32),
        pltpu.VMEM((CHE,), jnp.int32),
        pltpu.VMEM((CHE,), jnp.int32),
        pltpu.VMEM((CHE,), jnp.float32),
        pltpu.VMEM((CHE,), jnp.float32),
        pltpu.VMEM((CHE,), jnp.float32),
        pltpu.VMEM((CHE,), jnp.float32),
        pltpu.VMEM((N,), jnp.float32),
        pltpu.VMEM_SHARED((NPAD,), jnp.float32),
        pltpu.SemaphoreType.DMA,
        pltpu.SemaphoreType.DMA,
        pltpu.SemaphoreType.DMA,
        pltpu.SemaphoreType.DMA,
        pltpu.SemaphoreType.DMA,
        pltpu.SemaphoreType.DMA,
        pltpu.SemaphoreType.DMA,
        pltpu.SemaphoreType.DMA,
    ],
    compiler_params=_sc_params,
)
def _agg_kernel(edges_hbm, u_hbm, out_hbm,
                src0, src1, src2, src3, dst0, dst1, dst2, dst3,
                vals0, vals1, vals2, vals3, u_v, acc_sh,
                dma0, dma1, dma2, dma3, sc0, sc1, sc2, sc3):
    c = lax.axis_index("c")
    s = lax.axis_index("s")
    wid = c * 16 + s
    pltpu.sync_copy(u_hbm.at[pl.ds(0, N)], u_v)
    _fill(vals0, CHE, 0.0)
    _zero_acc_slice(vals0, acc_sh, s)
    plsc.subcore_barrier()

    base_e = wid * EPW
    nc = jnp.where(wid == 31, NCH_LAST, NCH_FULL)
    srcs = (src0, src1, src2, src3)
    dsts = (dst0, dst1, dst2, dst3)
    vals = (vals0, vals1, vals2, vals3)
    dmas = (dma0, dma1, dma2, dma3)
    scs = (sc0, sc1, sc2, sc3)

    def fire_dma(ci, P):
        e0 = base_e + ci * CHE
        pltpu.async_copy(edges_hbm.at[0, pl.ds(e0, CHE)], srcs[P], dmas[P])
        pltpu.async_copy(edges_hbm.at[1, pl.ds(e0, CHE)], dsts[P], dmas[P])

    def drain_sc(P):
        pltpu.make_async_copy(vals[P], acc_sh.at[dsts[P]], scs[P]).wait()

    def process(ci, P):
        Q = (P + 2) % 4
        pltpu.make_async_copy(edges_hbm.at[0, pl.ds(0, CHE)], srcs[P],
                              dmas[P]).wait()
        pltpu.make_async_copy(edges_hbm.at[1, pl.ds(0, CHE)], dsts[P],
                              dmas[P]).wait()

        @pl.when(ci >= 2)
        def _():
            drain_sc(Q)

        @pl.when(ci + 2 < nc)
        def _():
            fire_dma(ci + 2, Q)

        def gbody(i, _):
            for l in range(4):
                o = i * 64 + l * 16
                idx = srcs[P][pl.ds(o, 16)]
                vals[P][pl.ds(o, 16)] = plsc.load_gather(u_v, [idx])
            return 0

        lax.fori_loop(0, CHE // 64, gbody, 0)
        pltpu.async_copy(vals[P], acc_sh.at[dsts[P]], scs[P], add=True)

    fire_dma(0, 0)
    fire_dma(1, 1)

    def quad(k, carry):
        for t in range(4):
            process(k * 4 + t, t)
        return 0

    lax.fori_loop(0, nc // 4, quad, 0)
    for t in range(3):
        @pl.when(t < nc % 4)
        def _(t=t):
            process((nc // 4) * 4 + t, t)

    @pl.when(wid != 31)
    def _():
        drain_sc(0)
        drain_sc(1)

    @pl.when(wid == 31)
    def _():
        drain_sc(1)
        drain_sc(2)

    plsc.subcore_barrier()
    _writeback(vals0, acc_sh, out_hbm, c, s)


def _dinv_u1_body(degp_ref, x_ref, dinv_ref, u1_ref):
    deg = degp_ref[0] + degp_ref[1] + 1.0
    dinv = lax.rsqrt(deg)
    dinv_ref[...] = dinv
    u1_ref[...] = x_ref[...] * dinv


def _mid_body(t1p_ref, dinv_ref, x_ref, w1_ref, b1_ref, w2_ref,
              h2_ref, u2_ref):
    dinv = dinv_ref[...]
    a = dinv * (t1p_ref[0] + t1p_ref[1]) + x_ref[...] * dinv * dinv
    h2 = jnp.zeros_like(a)
    for j in range(3):
        h2 = h2 + w2_ref[0, j] * jnp.maximum(w1_ref[j, 0] * a + b1_ref[j], 0.0)
    h2_ref[...] = h2
    u2_ref[...] = h2 * dinv


def _final_body(t2p_ref, dinv_ref, h2_ref, b2_ref, out_ref):
    dinv = dinv_ref[...]
    o = dinv * (t2p_ref[0] + t2p_ref[1]) + h2_ref[...] * dinv * dinv + b2_ref[0]
    idx = (lax.broadcasted_iota(jnp.int32, o.shape, 0) * 128
           + lax.broadcasted_iota(jnp.int32, o.shape, 1))
    o = jnp.where(idx < N, o, -jnp.inf)
    m = jnp.max(o)
    e = jnp.exp(o - m)
    out_ref[...] = e / jnp.sum(e)


def _smem_spec():
    return pl.BlockSpec(memory_space=pltpu.SMEM)


def kernel(x, edge_index, W1, b1, W2, b2):
    x_pad = jnp.pad(x[:, 0], (0, NPAD - N)).reshape(784, 128)

    deg_p = _deg_kernel(edge_index)

    dinv, u1 = pl.pallas_call(
        _dinv_u1_body,
        out_shape=(jax.ShapeDtypeStruct((784, 128), jnp.float32),
                   jax.ShapeDtypeStruct((784, 128), jnp.float32)),
    )(deg_p.reshape(2, 784, 128), x_pad)

    t1_p = _agg_kernel(edge_index, u1.reshape(NPAD))

    h2, u2 = pl.pallas_call(
        _mid_body,
        in_specs=[pl.BlockSpec(), pl.BlockSpec(), pl.BlockSpec(),
                  _smem_spec(), _smem_spec(), _smem_spec()],
        out_shape=(jax.ShapeDtypeStruct((784, 128), jnp.float32),
                   jax.ShapeDtypeStruct((784, 128), jnp.float32)),
    )(t1_p.reshape(2, 784, 128), dinv, x_pad, W1, b1, W2)

    t2_p = _agg_kernel(edge_index, u2.reshape(NPAD))

    out = pl.pallas_call(
        _final_body,
        in_specs=[pl.BlockSpec(), pl.BlockSpec(), pl.BlockSpec(),
                  _smem_spec()],
        out_shape=jax.ShapeDtypeStruct((784, 128), jnp.float32),
    )(t2_p.reshape(2, 784, 128), dinv, h2, b2)

    return out.reshape(NPAD)[:N].reshape(N, 1)


# speedup vs baseline: 777.3615x; 1.0182x over previous
import functools

import jax
import jax.numpy as jnp
from jax import lax
from jax.experimental import pallas as pl
from jax.experimental.pallas import tpu as pltpu
from jax.experimental.pallas import tpu_sc as plsc

N = 100000
E = 6400000
NPAD = 100352
SL = NPAD // 16
CHE = 2048
EPW = 200704
NCH_FULL = EPW // CHE
NCH_LAST = (E - 31 * EPW) // CHE

_mesh = plsc.VectorSubcoreMesh(core_axis_name="c", subcore_axis_name="s")
_sc_params = pltpu.CompilerParams(needs_layout_passes=False)


def _fill(buf, n, value):
    def body(i, _):
        buf[pl.ds(i * 16, 16)] = jnp.full((16,), value, jnp.float32)
        return 0
    lax.fori_loop(0, n // 16, body, 0)


def _zero_acc_slice(zbuf, acc_sh, s):
    base = s * SL
    for i in range(3):
        pltpu.sync_copy(zbuf, acc_sh.at[pl.ds(base + i * CHE, CHE)])
    pltpu.sync_copy(zbuf.at[pl.ds(0, 128)], acc_sh.at[pl.ds(base + 3 * CHE, 128)])


def _writeback(tbuf, acc_sh, out_hbm, c, s):
    base = s * SL
    for i in range(3):
        pltpu.sync_copy(acc_sh.at[pl.ds(base + i * CHE, CHE)], tbuf)
        pltpu.sync_copy(tbuf, out_hbm.at[c, pl.ds(base + i * CHE, CHE)])
    pltpu.sync_copy(acc_sh.at[pl.ds(base + 3 * CHE, 128)], tbuf.at[pl.ds(0, 128)])
    pltpu.sync_copy(tbuf.at[pl.ds(0, 128)], out_hbm.at[c, pl.ds(base + 3 * CHE, 128)])


@functools.partial(
    pl.kernel,
    out_type=jax.ShapeDtypeStruct((2, NPAD), jnp.float32),
    mesh=_mesh,
    scratch_types=[
        pltpu.VMEM((CHE,), jnp.int32),
        pltpu.VMEM((CHE,), jnp.int32),
        pltpu.VMEM((CHE,), jnp.int32),
        pltpu.VMEM((CHE,), jnp.int32),
        pltpu.VMEM((CHE,), jnp.float32),
        pltpu.VMEM((CHE,), jnp.float32),
        pltpu.VMEM_SHARED((NPAD,), jnp.float32),
        pltpu.SemaphoreType.DMA,
        pltpu.SemaphoreType.DMA,
        pltpu.SemaphoreType.DMA,
        pltpu.SemaphoreType.DMA,
        pltpu.SemaphoreType.DMA,
        pltpu.SemaphoreType.DMA,
        pltpu.SemaphoreType.DMA,
        pltpu.SemaphoreType.DMA,
    ],
    compiler_params=_sc_params,
)
def _deg_kernel(edges_hbm, out_hbm, dst0, dst1, dst2, dst3, ones_v, zbuf,
                acc_sh, dma0, dma1, dma2, dma3, sc0, sc1, sc2, sc3):
    c = lax.axis_index("c")
    s = lax.axis_index("s")
    wid = c * 16 + s
    base_e = wid * EPW
    nc = jnp.where(wid == 31, NCH_LAST, NCH_FULL)
    dsts = (dst0, dst1, dst2, dst3)
    dmas = (dma0, dma1, dma2, dma3)
    scs = (sc0, sc1, sc2, sc3)

    def fire_dma(ci, P):
        e0 = base_e + ci * CHE
        pltpu.async_copy(edges_hbm.at[1, pl.ds(e0, CHE)], dsts[P], dmas[P])

    fire_dma(0, 0)
    fire_dma(1, 1)
    _fill(ones_v, CHE, 1.0)
    _fill(zbuf, CHE, 0.0)
    _zero_acc_slice(zbuf, acc_sh, s)
    plsc.subcore_barrier()

    def drain_sc(P):
        pltpu.make_async_copy(ones_v, acc_sh.at[dsts[P]], scs[P]).wait()

    def process(ci, P):
        Q = (P + 2) % 4
        pltpu.make_async_copy(edges_hbm.at[1, pl.ds(0, CHE)], dsts[P],
                              dmas[P]).wait()

        @pl.when(ci >= 2)
        def _():
            drain_sc(Q)

        @pl.when(ci + 2 < nc)
        def _():
            fire_dma(ci + 2, Q)

        pltpu.async_copy(ones_v, acc_sh.at[dsts[P]], scs[P], add=True)

    def quad(k, carry):
        for t in range(4):
            process(k * 4 + t, t)
        return 0

    lax.fori_loop(0, nc // 4, quad, 0)
    for t in range(3):
        @pl.when(t < nc % 4)
        def _(t=t):
            process((nc // 4) * 4 + t, t)

    @pl.when(wid != 31)
    def _():
        drain_sc(0)
        drain_sc(1)

    @pl.when(wid == 31)
    def _():
        drain_sc(1)
        drain_sc(2)

    plsc.subcore_barrier()
    _writeback(zbuf, acc_sh, out_hbm, c, s)


@functools.partial(
    pl.kernel,
    out_type=jax.ShapeDtypeStruct((2, NPAD), jnp.float32),
    mesh=_mesh,
    scratch_types=[
        pltpu.VMEM((CHE,), jnp.int32),
        pltpu.VMEM((CHE,), jnp.int32),
        pltpu.VMEM((CHE,), jnp.int32),
        pltpu.VMEM((CHE,), jnp.int32),
        pltpu.VMEM((CHE,), jnp.int32),
        pltpu.VMEM((CHE,), jnp.int32),
        pltpu.VMEM((CHE,), jnp.int32),
        pltpu.VMEM((CHE,), jnp.int32),
        pltpu.VMEM((CHE,), jnp.float32),
        pltpu.VMEM((CHE,), jnp.float32),
        pltpu.VMEM((CHE,), jnp.float32),
        pltpu.VMEM((CHE,), jnp.float32),
        pltpu.VMEM((N,), jnp.float32),
        pltpu.VMEM_SHARED((NPAD,), jnp.float32),
        pltpu.SemaphoreType.DMA,
        pltpu.SemaphoreType.DMA,
        pltpu.SemaphoreType.DMA,
        pltpu.SemaphoreType.DMA,
        pltpu.SemaphoreType.DMA,
        pltpu.SemaphoreType.DMA,
        pltpu.SemaphoreType.DMA,
        pltpu.SemaphoreType.DMA,
    ],
    compiler_params=_sc_params,
)
def _agg_kernel(edges_hbm, u_hbm, out_hbm,
                src0, src1, src2, src3, dst0, dst1, dst2, dst3,
                vals0, vals1, vals2, vals3, u_v, acc_sh,
                dma0, dma1, dma2, dma3, sc0, sc1, sc2, sc3):
    c = lax.axis_index("c")
    s = lax.axis_index("s")
    wid = c * 16 + s
    base_e = wid * EPW
    nc = jnp.where(wid == 31, NCH_LAST, NCH_FULL)
    srcs = (src0, src1, src2, src3)
    dsts = (dst0, dst1, dst2, dst3)
    vals = (vals0, vals1, vals2, vals3)
    dmas = (dma0, dma1, dma2, dma3)
    scs = (sc0, sc1, sc2, sc3)

    def fire_dma(ci, P):
        e0 = base_e + ci * CHE
        pltpu.async_copy(edges_hbm.at[0, pl.ds(e0, CHE)], srcs[P], dmas[P])
        pltpu.async_copy(edges_hbm.at[1, pl.ds(e0, CHE)], dsts[P], dmas[P])

    fire_dma(0, 0)
    fire_dma(1, 1)
    pltpu.async_copy(u_hbm.at[pl.ds(0, N)], u_v, sc3)
    _fill(vals0, CHE, 0.0)
    _zero_acc_slice(vals0, acc_sh, s)
    plsc.subcore_barrier()
    pltpu.make_async_copy(u_hbm.at[pl.ds(0, N)], u_v, sc3).wait()

    def drain_sc(P):
        pltpu.make_async_copy(vals[P], acc_sh.at[dsts[P]], scs[P]).wait()

    def process(ci, P):
        Q = (P + 2) % 4
        pltpu.make_async_copy(edges_hbm.at[0, pl.ds(0, CHE)], srcs[P],
                              dmas[P]).wait()
        pltpu.make_async_copy(edges_hbm.at[1, pl.ds(0, CHE)], dsts[P],
                              dmas[P]).wait()

        @pl.when(ci >= 2)
        def _():
            drain_sc(Q)

        @pl.when(ci + 2 < nc)
        def _():
            fire_dma(ci + 2, Q)

        def gbody(i, _):
            for l in range(8):
                o = i * 128 + l * 16
                idx = srcs[P][pl.ds(o, 16)]
                vals[P][pl.ds(o, 16)] = plsc.load_gather(u_v, [idx])
            return 0

        lax.fori_loop(0, CHE // 128, gbody, 0)
        pltpu.async_copy(vals[P], acc_sh.at[dsts[P]], scs[P], add=True)

    def quad(k, carry):
        for t in range(4):
            process(k * 4 + t, t)
        return 0

    lax.fori_loop(0, nc // 4, quad, 0)
    for t in range(3):
        @pl.when(t < nc % 4)
        def _(t=t):
            process((nc // 4) * 4 + t, t)

    @pl.when(wid != 31)
    def _():
        drain_sc(0)
        drain_sc(1)

    @pl.when(wid == 31)
    def _():
        drain_sc(1)
        drain_sc(2)

    plsc.subcore_barrier()
    _writeback(vals0, acc_sh, out_hbm, c, s)


def _dinv_u1_body(degp_ref, x_ref, dinv_ref, u1_ref):
    deg = degp_ref[0] + degp_ref[1] + 1.0
    dinv = lax.rsqrt(deg)
    dinv_ref[...] = dinv
    u1_ref[...] = x_ref[...] * dinv


def _mid_body(t1p_ref, dinv_ref, x_ref, w1_ref, b1_ref, w2_ref,
              h2_ref, u2_ref):
    dinv = dinv_ref[...]
    a = dinv * (t1p_ref[0] + t1p_ref[1]) + x_ref[...] * dinv * dinv
    h2 = jnp.zeros_like(a)
    for j in range(3):
        h2 = h2 + w2_ref[0, j] * jnp.maximum(w1_ref[j, 0] * a + b1_ref[j], 0.0)
    h2_ref[...] = h2
    u2_ref[...] = h2 * dinv


def _final_body(t2p_ref, dinv_ref, h2_ref, b2_ref, out_ref):
    dinv = dinv_ref[...]
    o = dinv * (t2p_ref[0] + t2p_ref[1]) + h2_ref[...] * dinv * dinv + b2_ref[0]
    idx = (lax.broadcasted_iota(jnp.int32, o.shape, 0) * 128
           + lax.broadcasted_iota(jnp.int32, o.shape, 1))
    o = jnp.where(idx < N, o, -jnp.inf)
    m = jnp.max(o)
    e = jnp.exp(o - m)
    out_ref[...] = e / jnp.sum(e)


def _smem_spec():
    return pl.BlockSpec(memory_space=pltpu.SMEM)


def kernel(x, edge_index, W1, b1, W2, b2):
    x_pad = jnp.pad(x[:, 0], (0, NPAD - N)).reshape(784, 128)

    deg_p = _deg_kernel(edge_index)

    dinv, u1 = pl.pallas_call(
        _dinv_u1_body,
        out_shape=(jax.ShapeDtypeStruct((784, 128), jnp.float32),
                   jax.ShapeDtypeStruct((784, 128), jnp.float32)),
    )(deg_p.reshape(2, 784, 128), x_pad)

    t1_p = _agg_kernel(edge_index, u1.reshape(NPAD))

    h2, u2 = pl.pallas_call(
        _mid_body,
        in_specs=[pl.BlockSpec(), pl.BlockSpec(), pl.BlockSpec(),
                  _smem_spec(), _smem_spec(), _smem_spec()],
        out_shape=(jax.ShapeDtypeStruct((784, 128), jnp.float32),
                   jax.ShapeDtypeStruct((784, 128), jnp.float32)),
    )(t1_p.reshape(2, 784, 128), dinv, x_pad, W1, b1, W2)

    t2_p = _agg_kernel(edge_index, u2.reshape(NPAD))

    out = pl.pallas_call(
        _final_body,
        in_specs=[pl.BlockSpec(), pl.BlockSpec(), pl.BlockSpec(),
                  _smem_spec()],
        out_shape=jax.ShapeDtypeStruct((784, 128), jnp.float32),
    )(t2_p.reshape(2, 784, 128), dinv, h2, b2)

    return out.reshape(NPAD)[:N].reshape(N, 1)


# speedup vs baseline: 778.3512x; 1.0013x over previous
import functools

import jax
import jax.numpy as jnp
from jax import lax
from jax.experimental import pallas as pl
from jax.experimental.pallas import tpu as pltpu
from jax.experimental.pallas import tpu_sc as plsc

N = 100000
E = 6400000
NPAD = 100352
SL = NPAD // 16
CHE = 2048
EPW = 200704
NCH_FULL = EPW // CHE
NCH_LAST = (E - 31 * EPW) // CHE

_mesh = plsc.VectorSubcoreMesh(core_axis_name="c", subcore_axis_name="s")
_sc_params = pltpu.CompilerParams(needs_layout_passes=False)


def _fill(buf, n, value):
    def body(i, _):
        buf[pl.ds(i * 16, 16)] = jnp.full((16,), value, jnp.float32)
        return 0
    lax.fori_loop(0, n // 16, body, 0)


def _zero_acc_slice(zbuf, acc_sh, s):
    base = s * SL
    for i in range(3):
        pltpu.sync_copy(zbuf, acc_sh.at[pl.ds(base + i * CHE, CHE)])
    pltpu.sync_copy(zbuf.at[pl.ds(0, 128)], acc_sh.at[pl.ds(base + 3 * CHE, 128)])


def _writeback(tbuf, acc_sh, out_hbm, c, s):
    base = s * SL
    for i in range(3):
        pltpu.sync_copy(acc_sh.at[pl.ds(base + i * CHE, CHE)], tbuf)
        pltpu.sync_copy(tbuf, out_hbm.at[c, pl.ds(base + i * CHE, CHE)])
    pltpu.sync_copy(acc_sh.at[pl.ds(base + 3 * CHE, 128)], tbuf.at[pl.ds(0, 128)])
    pltpu.sync_copy(tbuf.at[pl.ds(0, 128)], out_hbm.at[c, pl.ds(base + 3 * CHE, 128)])


@functools.partial(
    pl.kernel,
    out_type=jax.ShapeDtypeStruct((2, NPAD), jnp.float32),
    mesh=_mesh,
    scratch_types=[
        pltpu.VMEM((CHE,), jnp.int32),
        pltpu.VMEM((CHE,), jnp.int32),
        pltpu.VMEM((CHE,), jnp.int32),
        pltpu.VMEM((CHE,), jnp.int32),
        pltpu.VMEM((CHE,), jnp.float32),
        pltpu.VMEM((CHE,), jnp.float32),
        pltpu.VMEM_SHARED((NPAD,), jnp.float32),
        pltpu.SemaphoreType.DMA,
        pltpu.SemaphoreType.DMA,
        pltpu.SemaphoreType.DMA,
        pltpu.SemaphoreType.DMA,
        pltpu.SemaphoreType.DMA,
        pltpu.SemaphoreType.DMA,
        pltpu.SemaphoreType.DMA,
        pltpu.SemaphoreType.DMA,
    ],
    compiler_params=_sc_params,
)
def _deg_kernel(edges_hbm, out_hbm, dst0, dst1, dst2, dst3, ones_v, zbuf,
                acc_sh, dma0, dma1, dma2, dma3, sc0, sc1, sc2, sc3):
    c = lax.axis_index("c")
    s = lax.axis_index("s")
    wid = c * 16 + s
    base_e = wid * EPW
    nc = jnp.where(wid == 31, NCH_LAST, NCH_FULL)
    dsts = (dst0, dst1, dst2, dst3)
    dmas = (dma0, dma1, dma2, dma3)
    scs = (sc0, sc1, sc2, sc3)

    def fire_dma(ci, P):
        e0 = base_e + ci * CHE
        pltpu.async_copy(edges_hbm.at[1, pl.ds(e0, CHE)], dsts[P], dmas[P])

    fire_dma(0, 0)
    fire_dma(1, 1)
    _fill(ones_v, CHE, 1.0)
    _fill(zbuf, CHE, 0.0)
    _zero_acc_slice(zbuf, acc_sh, s)
    plsc.subcore_barrier()

    def drain_sc(P):
        pltpu.make_async_copy(ones_v, acc_sh.at[dsts[P]], scs[P]).wait()

    def process(ci, P):
        Q = (P + 2) % 4
        pltpu.make_async_copy(edges_hbm.at[1, pl.ds(0, CHE)], dsts[P],
                              dmas[P]).wait()

        @pl.when(ci >= 2)
        def _():
            drain_sc(Q)

        @pl.when(ci + 2 < nc)
        def _():
            fire_dma(ci + 2, Q)

        pltpu.async_copy(ones_v, acc_sh.at[dsts[P]], scs[P], add=True)

    def quad(k, carry):
        for t in range(4):
            process(k * 4 + t, t)
        return 0

    lax.fori_loop(0, nc // 4, quad, 0)
    for t in range(3):
        @pl.when(t < nc % 4)
        def _(t=t):
            process((nc // 4) * 4 + t, t)

    @pl.when(wid != 31)
    def _():
        drain_sc(0)
        drain_sc(1)

    @pl.when(wid == 31)
    def _():
        drain_sc(1)
        drain_sc(2)

    plsc.subcore_barrier()
    _writeback(zbuf, acc_sh, out_hbm, c, s)


@functools.partial(
    pl.kernel,
    out_type=jax.ShapeDtypeStruct((2, NPAD), jnp.float32),
    mesh=_mesh,
    scratch_types=[
        pltpu.VMEM((CHE,), jnp.int32),
        pltpu.VMEM((CHE,), jnp.int32),
        pltpu.VMEM((CHE,), jnp.int32),
        pltpu.VMEM((CHE,), jnp.int32),
        pltpu.VMEM((CHE,), jnp.int32),
        pltpu.VMEM((CHE,), jnp.int32),
        pltpu.VMEM((CHE,), jnp.int32),
        pltpu.VMEM((CHE,), jnp.int32),
        pltpu.VMEM((CHE,), jnp.float32),
        pltpu.VMEM((CHE,), jnp.float32),
        pltpu.VMEM((CHE,), jnp.float32),
        pltpu.VMEM((CHE,), jnp.float32),
        pltpu.VMEM((N,), jnp.float32),
        pltpu.VMEM_SHARED((NPAD,), jnp.float32),
        pltpu.SemaphoreType.DMA,
        pltpu.SemaphoreType.DMA,
        pltpu.SemaphoreType.DMA,
        pltpu.SemaphoreType.DMA,
        pltpu.SemaphoreType.DMA,
        pltpu.SemaphoreType.DMA,
        pltpu.SemaphoreType.DMA,
        pltpu.SemaphoreType.DMA,
    ],
    compiler_params=_sc_params,
)
def _agg_kernel(edges_hbm, u_hbm, out_hbm,
                src0, src1, src2, src3, dst0, dst1, dst2, dst3,
                vals0, vals1, vals2, vals3, u_v, acc_sh,
                dma0, dma1, dma2, dma3, sc0, sc1, sc2, sc3):
    c = lax.axis_index("c")
    s = lax.axis_index("s")
    wid = c * 16 + s
    base_e = wid * EPW
    nc = jnp.where(wid == 31, NCH_LAST, NCH_FULL)
    srcs = (src0, src1, src2, src3)
    dsts = (dst0, dst1, dst2, dst3)
    vals = (vals0, vals1, vals2, vals3)
    dmas = (dma0, dma1, dma2, dma3)
    scs = (sc0, sc1, sc2, sc3)

    def fire_dma(ci, P):
        e0 = base_e + ci * CHE
        pltpu.async_copy(edges_hbm.at[0, pl.ds(e0, CHE)], srcs[P], dmas[P])
        pltpu.async_copy(edges_hbm.at[1, pl.ds(e0, CHE)], dsts[P], dmas[P])

    fire_dma(0, 0)
    fire_dma(1, 1)
    pltpu.async_copy(u_hbm.at[pl.ds(0, N)], u_v, sc3)
    _fill(vals0, CHE, 0.0)
    _zero_acc_slice(vals0, acc_sh, s)
    plsc.subcore_barrier()
    pltpu.make_async_copy(u_hbm.at[pl.ds(0, N)], u_v, sc3).wait()

    def drain_sc(P):
        pltpu.make_async_copy(vals[P], acc_sh.at[dsts[P]], scs[P]).wait()

    def process(ci, P):
        Q = (P + 2) % 4
        pltpu.make_async_copy(edges_hbm.at[0, pl.ds(0, CHE)], srcs[P],
                              dmas[P]).wait()
        pltpu.make_async_copy(edges_hbm.at[1, pl.ds(0, CHE)], dsts[P],
                              dmas[P]).wait()

        @pl.when(ci >= 2)
        def _():
            drain_sc(Q)

        @pl.when(ci + 2 < nc)
        def _():
            fire_dma(ci + 2, Q)

        @plsc.parallel_loop(0, CHE // 16, unroll=8)
        def _(i):
            o = i * 16
            idx = srcs[P][pl.ds(o, 16)]
            vals[P][pl.ds(o, 16)] = plsc.load_gather(u_v, [idx])
        pltpu.async_copy(vals[P], acc_sh.at[dsts[P]], scs[P], add=True)

    def quad(k, carry):
        for t in range(4):
            process(k * 4 + t, t)
        return 0

    lax.fori_loop(0, nc // 4, quad, 0)
    for t in range(3):
        @pl.when(t < nc % 4)
        def _(t=t):
            process((nc // 4) * 4 + t, t)

    @pl.when(wid != 31)
    def _():
        drain_sc(0)
        drain_sc(1)

    @pl.when(wid == 31)
    def _():
        drain_sc(1)
        drain_sc(2)

    plsc.subcore_barrier()
    _writeback(vals0, acc_sh, out_hbm, c, s)


def _dinv_u1_body(degp_ref, x_ref, dinv_ref, u1_ref):
    deg = degp_ref[0] + degp_ref[1] + 1.0
    dinv = lax.rsqrt(deg)
    dinv_ref[...] = dinv
    u1_ref[...] = x_ref[...] * dinv


def _mid_body(t1p_ref, dinv_ref, x_ref, w1_ref, b1_ref, w2_ref,
              h2_ref, u2_ref):
    dinv = dinv_ref[...]
    a = dinv * (t1p_ref[0] + t1p_ref[1]) + x_ref[...] * dinv * dinv
    h2 = jnp.zeros_like(a)
    for j in range(3):
        h2 = h2 + w2_ref[0, j] * jnp.maximum(w1_ref[j, 0] * a + b1_ref[j], 0.0)
    h2_ref[...] = h2
    u2_ref[...] = h2 * dinv


def _final_body(t2p_ref, dinv_ref, h2_ref, b2_ref, out_ref):
    dinv = dinv_ref[...]
    o = dinv * (t2p_ref[0] + t2p_ref[1]) + h2_ref[...] * dinv * dinv + b2_ref[0]
    idx = (lax.broadcasted_iota(jnp.int32, o.shape, 0) * 128
           + lax.broadcasted_iota(jnp.int32, o.shape, 1))
    o = jnp.where(idx < N, o, -jnp.inf)
    m = jnp.max(o)
    e = jnp.exp(o - m)
    out_ref[...] = e / jnp.sum(e)


def _smem_spec():
    return pl.BlockSpec(memory_space=pltpu.SMEM)


def kernel(x, edge_index, W1, b1, W2, b2):
    x_pad = jnp.pad(x[:, 0], (0, NPAD - N)).reshape(784, 128)

    deg_p = _deg_kernel(edge_index)

    dinv, u1 = pl.pallas_call(
        _dinv_u1_body,
        out_shape=(jax.ShapeDtypeStruct((784, 128), jnp.float32),
                   jax.ShapeDtypeStruct((784, 128), jnp.float32)),
    )(deg_p.reshape(2, 784, 128), x_pad)

    t1_p = _agg_kernel(edge_index, u1.reshape(NPAD))

    h2, u2 = pl.pallas_call(
        _mid_body,
        in_specs=[pl.BlockSpec(), pl.BlockSpec(), pl.BlockSpec(),
                  _smem_spec(), _smem_spec(), _smem_spec()],
        out_shape=(jax.ShapeDtypeStruct((784, 128), jnp.float32),
                   jax.ShapeDtypeStruct((784, 128), jnp.float32)),
    )(t1_p.reshape(2, 784, 128), dinv, x_pad, W1, b1, W2)

    t2_p = _agg_kernel(edge_index, u2.reshape(NPAD))

    out = pl.pallas_call(
        _final_body,
        in_specs=[pl.BlockSpec(), pl.BlockSpec(), pl.BlockSpec(),
                  _smem_spec()],
        out_shape=jax.ShapeDtypeStruct((784, 128), jnp.float32),
    )(t2_p.reshape(2, 784, 128), dinv, h2, b2)

    return out.reshape(NPAD)[:N].reshape(N, 1)


# speedup vs baseline: 790.0718x; 1.0151x over previous
import functools

import jax
import jax.numpy as jnp
from jax import lax
from jax.experimental import pallas as pl
from jax.experimental.pallas import tpu as pltpu
from jax.experimental.pallas import tpu_sc as plsc

N = 100000
E = 6400000
NPAD = 100352
SL = NPAD // 16
CHE = 2048
EPW = 200704
NCH_FULL = EPW // CHE
NCH_LAST = (E - 31 * EPW) // CHE

_mesh = plsc.VectorSubcoreMesh(core_axis_name="c", subcore_axis_name="s")
_sc_params = pltpu.CompilerParams(needs_layout_passes=False)


def _fill(buf, n, value):
    def body(i, _):
        buf[pl.ds(i * 16, 16)] = jnp.full((16,), value, jnp.float32)
        return 0
    lax.fori_loop(0, n // 16, body, 0)


def _zero_acc_slice(zbuf, acc_sh, s):
    base = s * SL
    for i in range(3):
        pltpu.sync_copy(zbuf, acc_sh.at[pl.ds(base + i * CHE, CHE)])
    pltpu.sync_copy(zbuf.at[pl.ds(0, 128)], acc_sh.at[pl.ds(base + 3 * CHE, 128)])


def _writeback(tbuf, acc_sh, out_hbm, c, s):
    base = s * SL
    for i in range(3):
        pltpu.sync_copy(acc_sh.at[pl.ds(base + i * CHE, CHE)], tbuf)
        pltpu.sync_copy(tbuf, out_hbm.at[c, pl.ds(base + i * CHE, CHE)])
    pltpu.sync_copy(acc_sh.at[pl.ds(base + 3 * CHE, 128)], tbuf.at[pl.ds(0, 128)])
    pltpu.sync_copy(tbuf.at[pl.ds(0, 128)], out_hbm.at[c, pl.ds(base + 3 * CHE, 128)])


@functools.partial(
    pl.kernel,
    out_type=jax.ShapeDtypeStruct((2, NPAD), jnp.float32),
    mesh=_mesh,
    scratch_types=[
        pltpu.VMEM((CHE,), jnp.int32),
        pltpu.VMEM((CHE,), jnp.int32),
        pltpu.VMEM((CHE,), jnp.int32),
        pltpu.VMEM((CHE,), jnp.int32),
        pltpu.VMEM((CHE,), jnp.float32),
        pltpu.VMEM((CHE,), jnp.float32),
        pltpu.VMEM_SHARED((NPAD,), jnp.float32),
        pltpu.SemaphoreType.DMA,
        pltpu.SemaphoreType.DMA,
        pltpu.SemaphoreType.DMA,
        pltpu.SemaphoreType.DMA,
        pltpu.SemaphoreType.DMA,
        pltpu.SemaphoreType.DMA,
        pltpu.SemaphoreType.DMA,
        pltpu.SemaphoreType.DMA,
    ],
    compiler_params=_sc_params,
)
def _deg_kernel(edges_hbm, out_hbm, dst0, dst1, dst2, dst3, ones_v, zbuf,
                acc_sh, dma0, dma1, dma2, dma3, sc0, sc1, sc2, sc3):
    c = lax.axis_index("c")
    s = lax.axis_index("s")
    wid = c * 16 + s
    base_e = wid * EPW
    nc = jnp.where(wid == 31, NCH_LAST, NCH_FULL)
    dsts = (dst0, dst1, dst2, dst3)
    dmas = (dma0, dma1, dma2, dma3)
    scs = (sc0, sc1, sc2, sc3)

    def fire_dma(ci, P):
        e0 = base_e + ci * CHE
        pltpu.async_copy(edges_hbm.at[1, pl.ds(e0, CHE)], dsts[P], dmas[P])

    fire_dma(0, 0)
    fire_dma(1, 1)
    _fill(ones_v, CHE, 1.0)
    _fill(zbuf, CHE, 0.0)
    _zero_acc_slice(zbuf, acc_sh, s)
    plsc.subcore_barrier()

    def drain_sc(P):
        pltpu.make_async_copy(ones_v, acc_sh.at[dsts[P]], scs[P]).wait()

    def process(ci, P):
        Q = (P + 2) % 4
        pltpu.make_async_copy(edges_hbm.at[1, pl.ds(0, CHE)], dsts[P],
                              dmas[P]).wait()

        @pl.when(ci >= 2)
        def _():
            drain_sc(Q)

        @pl.when(ci + 2 < nc)
        def _():
            fire_dma(ci + 2, Q)

        pltpu.async_copy(ones_v, acc_sh.at[dsts[P]], scs[P], add=True)

    def quad(k, carry):
        for t in range(4):
            process(k * 4 + t, t)
        return 0

    lax.fori_loop(0, nc // 4, quad, 0)
    for t in range(3):
        @pl.when(t < nc % 4)
        def _(t=t):
            process((nc // 4) * 4 + t, t)

    @pl.when(wid != 31)
    def _():
        drain_sc(0)
        drain_sc(1)

    @pl.when(wid == 31)
    def _():
        drain_sc(1)
        drain_sc(2)

    plsc.subcore_barrier()
    _writeback(zbuf, acc_sh, out_hbm, c, s)


@functools.partial(
    pl.kernel,
    out_type=jax.ShapeDtypeStruct((2, NPAD), jnp.float32),
    mesh=_mesh,
    scratch_types=[
        pltpu.VMEM((CHE,), jnp.int32),
        pltpu.VMEM((CHE,), jnp.int32),
        pltpu.VMEM((CHE,), jnp.int32),
        pltpu.VMEM((CHE,), jnp.int32),
        pltpu.VMEM((CHE,), jnp.int32),
        pltpu.VMEM((CHE,), jnp.int32),
        pltpu.VMEM((CHE,), jnp.int32),
        pltpu.VMEM((CHE,), jnp.int32),
        pltpu.VMEM((CHE,), jnp.float32),
        pltpu.VMEM((CHE,), jnp.float32),
        pltpu.VMEM((CHE,), jnp.float32),
        pltpu.VMEM((CHE,), jnp.float32),
        pltpu.VMEM((N,), jnp.float32),
        pltpu.VMEM_SHARED((NPAD,), jnp.float32),
        pltpu.SemaphoreType.DMA,
        pltpu.SemaphoreType.DMA,
        pltpu.SemaphoreType.DMA,
        pltpu.SemaphoreType.DMA,
        pltpu.SemaphoreType.DMA,
        pltpu.SemaphoreType.DMA,
        pltpu.SemaphoreType.DMA,
        pltpu.SemaphoreType.DMA,
    ],
    compiler_params=_sc_params,
)
def _agg_kernel(edges_hbm, u_hbm, out_hbm,
                src0, src1, src2, src3, dst0, dst1, dst2, dst3,
                vals0, vals1, vals2, vals3, u_v, acc_sh,
                dma0, dma1, dma2, dma3, sc0, sc1, sc2, sc3):
    c = lax.axis_index("c")
    s = lax.axis_index("s")
    wid = c * 16 + s
    base_e = wid * EPW
    nc = jnp.where(wid == 31, NCH_LAST, NCH_FULL)
    srcs = (src0, src1, src2, src3)
    dsts = (dst0, dst1, dst2, dst3)
    vals = (vals0, vals1, vals2, vals3)
    dmas = (dma0, dma1, dma2, dma3)
    scs = (sc0, sc1, sc2, sc3)

    def fire_dma(ci, P):
        e0 = base_e + ci * CHE
        pltpu.async_copy(edges_hbm.at[0, pl.ds(e0, CHE)], srcs[P], dmas[P])
        pltpu.async_copy(edges_hbm.at[1, pl.ds(e0, CHE)], dsts[P], dmas[P])

    fire_dma(0, 0)
    fire_dma(1, 1)
    pltpu.async_copy(u_hbm.at[pl.ds(0, N)], u_v, sc3)
    _fill(vals0, CHE, 0.0)
    _zero_acc_slice(vals0, acc_sh, s)
    plsc.subcore_barrier()
    pltpu.make_async_copy(u_hbm.at[pl.ds(0, N)], u_v, sc3).wait()

    def drain_sc(P):
        pltpu.make_async_copy(vals[P], acc_sh.at[dsts[P]], scs[P]).wait()

    def process(ci, P):
        Q = (P + 2) % 4
        pltpu.make_async_copy(edges_hbm.at[0, pl.ds(0, CHE)], srcs[P],
                              dmas[P]).wait()
        pltpu.make_async_copy(edges_hbm.at[1, pl.ds(0, CHE)], dsts[P],
                              dmas[P]).wait()

        @plsc.parallel_loop(0, CHE // 16, unroll=8)
        def _(i):
            o = i * 16
            idx = srcs[P][pl.ds(o, 16)]
            vals[P][pl.ds(o, 16)] = plsc.load_gather(u_v, [idx])

        @pl.when(ci >= 2)
        def _():
            drain_sc(Q)

        @pl.when(ci + 2 < nc)
        def _():
            fire_dma(ci + 2, Q)

        pltpu.async_copy(vals[P], acc_sh.at[dsts[P]], scs[P], add=True)

    def quad(k, carry):
        for t in range(4):
            process(k * 4 + t, t)
        return 0

    lax.fori_loop(0, nc // 4, quad, 0)
    for t in range(3):
        @pl.when(t < nc % 4)
        def _(t=t):
            process((nc // 4) * 4 + t, t)

    @pl.when(wid != 31)
    def _():
        drain_sc(0)
        drain_sc(1)

    @pl.when(wid == 31)
    def _():
        drain_sc(1)
        drain_sc(2)

    plsc.subcore_barrier()
    _writeback(vals0, acc_sh, out_hbm, c, s)


def _dinv_u1_body(degp_ref, x_ref, dinv_ref, u1_ref):
    deg = degp_ref[0] + degp_ref[1] + 1.0
    dinv = lax.rsqrt(deg)
    dinv_ref[...] = dinv
    u1_ref[...] = x_ref[...] * dinv


def _mid_body(t1p_ref, dinv_ref, x_ref, w1_ref, b1_ref, w2_ref,
              h2_ref, u2_ref):
    dinv = dinv_ref[...]
    a = dinv * (t1p_ref[0] + t1p_ref[1]) + x_ref[...] * dinv * dinv
    h2 = jnp.zeros_like(a)
    for j in range(3):
        h2 = h2 + w2_ref[0, j] * jnp.maximum(w1_ref[j, 0] * a + b1_ref[j], 0.0)
    h2_ref[...] = h2
    u2_ref[...] = h2 * dinv


def _final_body(t2p_ref, dinv_ref, h2_ref, b2_ref, out_ref):
    dinv = dinv_ref[...]
    o = dinv * (t2p_ref[0] + t2p_ref[1]) + h2_ref[...] * dinv * dinv + b2_ref[0]
    idx = (lax.broadcasted_iota(jnp.int32, o.shape, 0) * 128
           + lax.broadcasted_iota(jnp.int32, o.shape, 1))
    o = jnp.where(idx < N, o, -jnp.inf)
    m = jnp.max(o)
    e = jnp.exp(o - m)
    out_ref[...] = e / jnp.sum(e)


def _smem_spec():
    return pl.BlockSpec(memory_space=pltpu.SMEM)


def kernel(x, edge_index, W1, b1, W2, b2):
    x_pad = jnp.pad(x[:, 0], (0, NPAD - N)).reshape(784, 128)

    deg_p = _deg_kernel(edge_index)

    dinv, u1 = pl.pallas_call(
        _dinv_u1_body,
        out_shape=(jax.ShapeDtypeStruct((784, 128), jnp.float32),
                   jax.ShapeDtypeStruct((784, 128), jnp.float32)),
    )(deg_p.reshape(2, 784, 128), x_pad)

    t1_p = _agg_kernel(edge_index, u1.reshape(NPAD))

    h2, u2 = pl.pallas_call(
        _mid_body,
        in_specs=[pl.BlockSpec(), pl.BlockSpec(), pl.BlockSpec(),
                  _smem_spec(), _smem_spec(), _smem_spec()],
        out_shape=(jax.ShapeDtypeStruct((784, 128), jnp.float32),
                   jax.ShapeDtypeStruct((784, 128), jnp.float32)),
    )(t1_p.reshape(2, 784, 128), dinv, x_pad, W1, b1, W2)

    t2_p = _agg_kernel(edge_index, u2.reshape(NPAD))

    out = pl.pallas_call(
        _final_body,
        in_specs=[pl.BlockSpec(), pl.BlockSpec(), pl.BlockSpec(),
                  _smem_spec()],
        out_shape=jax.ShapeDtypeStruct((784, 128), jnp.float32),
    )(t2_p.reshape(2, 784, 128), dinv, h2, b2)

    return out.reshape(NPAD)[:N].reshape(N, 1)
